# Initial kernel scaffold; baseline (speedup 1.0000x reference)
import jax
import jax.numpy as jnp
from jax.experimental import pallas as pl


def kernel(user, recipe, user_table, recipe_table, W1, b1, W2, b2, W3, b3):
    raise NotImplementedError("write your pallas kernel here")


# speedup vs baseline: 2.6801x; 2.6801x over previous
import functools

import jax
import jax.numpy as jnp
from jax import lax
from jax.experimental import pallas as pl
from jax.experimental.pallas import tpu as pltpu
from jax.experimental.pallas import tpu_sc as plsc

_B = 16384
_D = 128
_NC, _NS = 2, 16
_NW = _NC * _NS
_BPW = _B // _NW
_CHUNK = 128
_NCH = _BPW // _CHUNK

@functools.cache
def _make_sc_gather():
    mesh = plsc.VectorSubcoreMesh(core_axis_name="c", subcore_axis_name="s",
                                  num_cores=_NC, num_subcores=_NS)

    @functools.partial(
        pl.kernel,
        out_type=(
            jax.ShapeDtypeStruct((_NW, _BPW, _D), jnp.float32),
            jax.ShapeDtypeStruct((_NW, _BPW, _D), jnp.float32),
        ),
        mesh=mesh,
        scratch_types=[
            pltpu.VMEM((_NCH, _CHUNK), jnp.int32),
            pltpu.VMEM((_NCH, _CHUNK), jnp.int32),
            pltpu.VMEM((_BPW, _D), jnp.float32),
            pltpu.SemaphoreType.DMA,
        ],
    )
    def _sc_gather(uidx_hbm, ridx_hbm, utab_hbm, rtab_hbm, uout_hbm,
                   rout_hbm, uidx_v, ridx_v, rows_v, sem):
        wid = lax.axis_index("s") * _NC + lax.axis_index("c")
        pltpu.sync_copy(uidx_hbm.at[wid], uidx_v)
        pltpu.sync_copy(ridx_hbm.at[wid], ridx_v)
        descs = [
            pltpu.async_copy(utab_hbm.at[uidx_v.at[j]],
                             rows_v.at[pl.ds(j * _CHUNK, _CHUNK)], sem)
            for j in range(_NCH)
        ]
        for d in descs:
            d.wait()
        pltpu.sync_copy(rows_v, uout_hbm.at[wid])
        descs = [
            pltpu.async_copy(rtab_hbm.at[ridx_v.at[j]],
                             rows_v.at[pl.ds(j * _CHUNK, _CHUNK)], sem)
            for j in range(_NCH)
        ]
        for d in descs:
            d.wait()
        pltpu.sync_copy(rows_v, rout_hbm.at[wid])

    return _sc_gather


_BB = 1024


def _mlp_body(r_ref, u_ref, w1_ref, b1_ref, w2_ref, b2_ref, w3_ref, b3_ref,
              o_ref):
    w1 = w1_ref[...]
    h = jnp.dot(r_ref[...], w1[:_D], preferred_element_type=jnp.float32)
    h = h + jnp.dot(u_ref[...], w1[_D:], preferred_element_type=jnp.float32)
    h = jnp.maximum(h + b1_ref[...], 0.0)
    h = jnp.dot(h, w2_ref[...], preferred_element_type=jnp.float32)
    h = jnp.maximum(h + b2_ref[...], 0.0)
    o_ref[...] = (jnp.dot(h, w3_ref[...], preferred_element_type=jnp.float32)
                  + b3_ref[...])


def _mlp(r_emb, u_emb, W1, b1, W2, b2, W3, b3):
    return pl.pallas_call(
        _mlp_body,
        grid=(_B // _BB,),
        in_specs=[
            pl.BlockSpec((_BB, _D), lambda i: (i, 0)),
            pl.BlockSpec((_BB, _D), lambda i: (i, 0)),
            pl.BlockSpec((2 * _D, 100), lambda i: (0, 0)),
            pl.BlockSpec((1, 100), lambda i: (0, 0)),
            pl.BlockSpec((100, 50), lambda i: (0, 0)),
            pl.BlockSpec((1, 50), lambda i: (0, 0)),
            pl.BlockSpec((50, 1), lambda i: (0, 0)),
            pl.BlockSpec((1, 1), lambda i: (0, 0)),
        ],
        out_specs=pl.BlockSpec((_BB, 1), lambda i: (i, 0)),
        out_shape=jax.ShapeDtypeStruct((_B, 1), jnp.float32),
    )(r_emb, u_emb, W1, b1.reshape(1, -1), W2, b2.reshape(1, -1), W3,
      b3.reshape(1, -1))


def kernel(user, recipe, user_table, recipe_table, W1, b1, W2, b2, W3, b3):
    uidx = user.astype(jnp.int32).reshape(_NW, _NCH, _CHUNK)
    ridx = recipe.astype(jnp.int32).reshape(_NW, _NCH, _CHUNK)
    u_emb, r_emb = _make_sc_gather()(uidx, ridx, user_table, recipe_table)
    out = _mlp(r_emb.reshape(_B, _D), u_emb.reshape(_B, _D),
               W1, b1, W2, b2, W3, b3)
    return out.reshape(_B)
